# baseline (device time: 141896 ns/iter reference)
import jax
import jax.numpy as jnp
from jax import lax
from jax.experimental import pallas as pl
from jax.experimental.pallas import tpu as pltpu

N_Z = 4
M = 2048
D = 2048
QTR = M // 4
CHUNK = QTR // N_Z
SUB = CHUNK // 2
DSUB = SUB // 2


def kernel(partial, resid, gamma):
    gamma2d = gamma.reshape(1, D)

    def body(partial_ref, resid_ref, gamma_ref, out_ref,
             rs_buf, resid_chunk,
             rs_send, rs_recv, ag_send, ag_recv,
             y_send, y_recv, x_send, x_recv, resid_sem):
        my_x = lax.axis_index("x")
        my_y = lax.axis_index("y")
        r = lax.axis_index("z")
        right = (r + 1) % N_Z
        left = (r + 3) % N_Z
        q = 2 * my_x + my_y
        qbase = q * QTR
        o = (r + 1) % N_Z
        obase = qbase + o * CHUNK

        y_dev = (my_x, 1 - my_y, r)
        x_dev = (1 - my_x, my_y, r)
        qpbase = (2 * my_x + (1 - my_y)) * QTR
        qxbase = (2 * (1 - my_x) + my_y) * QTR

        barrier_sem = pltpu.get_barrier_semaphore()
        for dev in ((my_x, my_y, left), (my_x, my_y, right),
                    y_dev, x_dev):
            pl.semaphore_signal(
                barrier_sem, inc=1,
                device_id=dev, device_id_type=pl.DeviceIdType.MESH,
            )
        pl.semaphore_wait(barrier_sem, 4)

        rcopy = pltpu.make_async_copy(
            resid_ref.at[pl.ds(obase, CHUNK), :], resid_chunk, resid_sem
        )
        rcopy.start()

        pending = []

        def rs_rdma(s, g, src):
            return pltpu.make_async_remote_copy(
                src_ref=src,
                dst_ref=rs_buf.at[s, pl.ds(g * SUB, SUB), :],
                send_sem=rs_send.at[s, g],
                recv_sem=rs_recv.at[s, g],
                device_id=(my_x, my_y, right),
                device_id_type=pl.DeviceIdType.MESH,
            )

        rs_descs = {}
        for g in range(2):
            d = rs_rdma(
                0, g,
                partial_ref.at[0, pl.ds(qbase + r * CHUNK + g * SUB, SUB), :],
            )
            d.start()
            rs_descs[(0, g)] = d

        def rs_chain(g):
            for s in range(N_Z - 1):
                c = (r + (N_Z - 1 - s)) % N_Z
                d = rs_descs[(s, g)]
                d.wait_recv()
                pending.append(d)
                rs_buf[s, pl.ds(g * SUB, SUB), :] = (
                    rs_buf[s, pl.ds(g * SUB, SUB), :]
                    + partial_ref[
                        0, pl.ds(qbase + c * CHUNK + g * SUB, SUB), :
                    ]
                )
                if s < N_Z - 2:
                    nd = rs_rdma(
                        s + 1, g, rs_buf.at[s, pl.ds(g * SUB, SUB), :]
                    )
                    nd.start()
                    rs_descs[(s + 1, g)] = nd

        def norm(g):
            gsl = pl.ds(g * SUB, SUB)
            y = rs_buf[N_Z - 2, gsl, :] + resid_chunk[gsl, :]
            ms = jnp.mean(y * y, axis=1, keepdims=True)
            out_ref[pl.ds(obase + g * SUB, SUB), :] = (
                y * lax.rsqrt(ms + 1e-6) * gamma_ref[0, :]
            )

        def remote(sl, send, recv, dev):
            return pltpu.make_async_remote_copy(
                src_ref=out_ref.at[sl, :],
                dst_ref=out_ref.at[sl, :],
                send_sem=send,
                recv_sem=recv,
                device_id=dev,
                device_id_type=pl.DeviceIdType.MESH,
            )

        x_descs = {}
        y_descs = {}
        ag_descs = {}

        def send_x(g, slot, sl):
            d = remote(sl, x_send.at[g, slot], x_recv.at[g, slot], x_dev)
            d.start()
            x_descs[(g, slot)] = d
            pending.append(d)

        def send_y(g, slot, sl):
            d = remote(sl, y_send.at[g, slot], y_recv.at[g, slot], y_dev)
            d.start()
            y_descs[(g, slot)] = d
            pending.append(d)

        def blk(g, b):
            return pl.ds(qbase + b * CHUNK + g * SUB, SUB)

        def dist_init(g):
            sl_own = pl.ds(obase + g * SUB, SUB)
            ag = remote(sl_own, ag_send.at[g, 0], ag_recv.at[g, 0],
                        (my_x, my_y, right))
            ag.start()
            ag_descs[(g, 0)] = ag
            send_y(g, 0, sl_own)
            send_x(g, 0, sl_own)

        def dist_step(g, s):
            ag_descs[(g, s)].wait_recv()
            pending.append(ag_descs[(g, s)])
            nb = (r + N_Z - s) % N_Z
            sl_nb = blk(g, nb)
            if s < N_Z - 2:
                ag = remote(sl_nb, ag_send.at[g, s + 1],
                            ag_recv.at[g, s + 1], (my_x, my_y, right))
                ag.start()
                ag_descs[(g, s + 1)] = ag
            send_y(g, s + 1, sl_nb)
            send_x(g, 2 * (s + 1), sl_nb)
            y_descs[(g, s)].wait_recv()
            yb = (r + N_Z + 1 - s) % N_Z
            send_x(g, 2 * s + 1,
                   pl.ds(qpbase + yb * CHUNK + g * SUB + DSUB, DSUB))

        rs_chain(0)
        rcopy.wait()
        norm(0)
        dist_init(0)
        rs_chain(1)
        norm(1)
        dist_init(1)

        for s in range(N_Z - 1):
            dist_step(0, s)
            dist_step(1, s)

        for g in range(2):
            y_descs[(g, N_Z - 1)].wait_recv()
            yb = (r + 2) % N_Z
            send_x(g, 2 * (N_Z - 1) + 1,
                   pl.ds(qpbase + yb * CHUNK + g * SUB + DSUB, DSUB))

        for k in range(N_Z):
            for g in range(2):
                x_descs[(g, 2 * k)].wait_recv()
                xb = (r + N_Z + 1 - k) % N_Z
                send_y(g, N_Z + k,
                       pl.ds(qxbase + xb * CHUNK + g * SUB, DSUB))

        for d in pending:
            d.wait_send()
        for k in range(N_Z):
            for g in range(2):
                x_descs[(g, 2 * k + 1)].wait_recv()
                y_descs[(g, N_Z + k)].wait_recv()

    return pl.pallas_call(
        body,
        out_shape=jax.ShapeDtypeStruct((M, D), jnp.float32),
        in_specs=[
            pl.BlockSpec(memory_space=pltpu.VMEM),
            pl.BlockSpec(memory_space=pltpu.MemorySpace.HBM),
            pl.BlockSpec(memory_space=pltpu.VMEM),
        ],
        out_specs=pl.BlockSpec(memory_space=pltpu.VMEM),
        scratch_shapes=[
            pltpu.VMEM((N_Z - 1, CHUNK, D), jnp.float32),
            pltpu.VMEM((CHUNK, D), jnp.float32),
            pltpu.SemaphoreType.DMA((N_Z - 1, 2)),
            pltpu.SemaphoreType.DMA((N_Z - 1, 2)),
            pltpu.SemaphoreType.DMA((2, N_Z - 1)),
            pltpu.SemaphoreType.DMA((2, N_Z - 1)),
            pltpu.SemaphoreType.DMA((2, 2 * N_Z)),
            pltpu.SemaphoreType.DMA((2, 2 * N_Z)),
            pltpu.SemaphoreType.DMA((2, 2 * N_Z)),
            pltpu.SemaphoreType.DMA((2, 2 * N_Z)),
            pltpu.SemaphoreType.DMA,
        ],
        compiler_params=pltpu.CompilerParams(
            collective_id=0, vmem_limit_bytes=100 * 1024 * 1024
        ),
    )(partial, resid, gamma2d)


# device time: 141149 ns/iter; 1.0053x vs baseline; 1.0053x over previous
import jax
import jax.numpy as jnp
from jax import lax
from jax.experimental import pallas as pl
from jax.experimental.pallas import tpu as pltpu

N_Z = 4
M = 2048
D = 2048
QTR = M // 4
CHUNK = QTR // N_Z
HALF = M // 2


def kernel(partial, resid, gamma):
    gamma2d = gamma.reshape(1, D)

    def body(partial_ref, resid_ref, gamma_ref, out_ref,
             rs_buf, resid_chunk,
             rs_send, rs_recv, ag_send, ag_recv,
             y_send, y_recv, x_send, x_recv, resid_sem):
        my_x = lax.axis_index("x")
        my_y = lax.axis_index("y")
        r = lax.axis_index("z")
        right = (r + 1) % N_Z
        left = (r + 3) % N_Z
        q = 2 * my_x + my_y
        qbase = q * QTR

        barrier_sem = pltpu.get_barrier_semaphore()
        for dev in ((my_x, my_y, left), (my_x, my_y, right),
                    (my_x, 1 - my_y, r), (1 - my_x, my_y, r)):
            pl.semaphore_signal(
                barrier_sem, inc=1,
                device_id=dev, device_id_type=pl.DeviceIdType.MESH,
            )
        pl.semaphore_wait(barrier_sem, 4)

        o = (r + 1) % N_Z
        obase = qbase + o * CHUNK
        rcopy = pltpu.make_async_copy(
            resid_ref.at[pl.ds(obase, CHUNK), :], resid_chunk, resid_sem
        )
        rcopy.start()

        SUB = CHUNK // 2

        def rs_rdma(s, h, src):
            return pltpu.make_async_remote_copy(
                src_ref=src,
                dst_ref=rs_buf.at[s, pl.ds(h * SUB, SUB), :],
                send_sem=rs_send.at[s, h],
                recv_sem=rs_recv.at[s, h],
                device_id=(my_x, my_y, right),
                device_id_type=pl.DeviceIdType.MESH,
            )

        pending = []
        rs_descs = {}
        for h in range(2):
            d = rs_rdma(
                0, h,
                partial_ref.at[0, pl.ds(qbase + r * CHUNK + h * SUB, SUB), :],
            )
            d.start()
            rs_descs[(0, h)] = d
        for s in range(N_Z - 1):
            c = (r + (N_Z - 1 - s)) % N_Z
            for h in range(2):
                d = rs_descs[(s, h)]
                d.wait_recv()
                pending.append(d)
                rs_buf[s, pl.ds(h * SUB, SUB), :] = (
                    rs_buf[s, pl.ds(h * SUB, SUB), :]
                    + partial_ref[
                        0, pl.ds(qbase + c * CHUNK + h * SUB, SUB), :
                    ]
                )
                if s < N_Z - 2:
                    nd = rs_rdma(
                        s + 1, h, rs_buf.at[s, pl.ds(h * SUB, SUB), :]
                    )
                    nd.start()
                    rs_descs[(s + 1, h)] = nd

        rcopy.wait()
        y = rs_buf[N_Z - 2, :, :] + resid_chunk[:, :]
        ms = jnp.mean(y * y, axis=1, keepdims=True)
        out_ref[pl.ds(obase, CHUNK), :] = (
            y * lax.rsqrt(ms + 1e-6) * gamma_ref[0, :]
        )

        y_dev = (my_x, 1 - my_y, r)
        x_dev = (1 - my_x, my_y, r)
        qpbase = (2 * my_x + (1 - my_y)) * QTR
        qxbase = (2 * (1 - my_x) + my_y) * QTR

        def remote(sl, send, recv, dev):
            return pltpu.make_async_remote_copy(
                src_ref=out_ref.at[sl, :],
                dst_ref=out_ref.at[sl, :],
                send_sem=send,
                recv_sem=recv,
                device_id=dev,
                device_id_type=pl.DeviceIdType.MESH,
            )

        x_descs = {}
        y_descs = {}

        def send_x(slot, sl):
            d = remote(sl, x_send.at[slot], x_recv.at[slot], x_dev)
            d.start()
            x_descs[slot] = d
            pending.append(d)

        def send_y(slot, sl):
            d = remote(sl, y_send.at[slot], y_recv.at[slot], y_dev)
            d.start()
            y_descs[slot] = d
            pending.append(d)

        sl_own = pl.ds(obase, CHUNK)
        ag = remote(sl_own, ag_send.at[0], ag_recv.at[0],
                    (my_x, my_y, right))
        ag.start()
        ag_descs = [ag]
        send_y(0, sl_own)
        send_x(0, sl_own)

        for s in range(N_Z - 1):
            ag_descs[s].wait_recv()
            pending.append(ag_descs[s])
            nb = (r + N_Z - s) % N_Z
            sl_nb = pl.ds(qbase + nb * CHUNK, CHUNK)
            if s < N_Z - 2:
                ag = remote(sl_nb, ag_send.at[s + 1], ag_recv.at[s + 1],
                            (my_x, my_y, right))
                ag.start()
                ag_descs.append(ag)
            send_y(s + 1, sl_nb)
            send_x(2 * (s + 1), sl_nb)
            y_descs[s].wait_recv()
            yb = (r + N_Z + 1 - s) % N_Z
            send_x(2 * s + 1, pl.ds(qpbase + yb * CHUNK + SUB, SUB))

        y_descs[N_Z - 1].wait_recv()
        yb = (r + 2) % N_Z
        send_x(2 * (N_Z - 1) + 1, pl.ds(qpbase + yb * CHUNK + SUB, SUB))

        for k in range(N_Z):
            x_descs[2 * k].wait_recv()
            xb = (r + N_Z + 1 - k) % N_Z
            send_y(N_Z + k, pl.ds(qxbase + xb * CHUNK, SUB))

        for d in pending:
            d.wait_send()
        for k in range(N_Z):
            x_descs[2 * k + 1].wait_recv()
            y_descs[N_Z + k].wait_recv()

    return pl.pallas_call(
        body,
        out_shape=jax.ShapeDtypeStruct((M, D), jnp.float32),
        in_specs=[
            pl.BlockSpec(memory_space=pltpu.VMEM),
            pl.BlockSpec(memory_space=pltpu.MemorySpace.HBM),
            pl.BlockSpec(memory_space=pltpu.VMEM),
        ],
        out_specs=pl.BlockSpec(memory_space=pltpu.VMEM),
        scratch_shapes=[
            pltpu.VMEM((N_Z - 1, CHUNK, D), jnp.float32),
            pltpu.VMEM((CHUNK, D), jnp.float32),
            pltpu.SemaphoreType.DMA((N_Z - 1, 2)),
            pltpu.SemaphoreType.DMA((N_Z - 1, 2)),
            pltpu.SemaphoreType.DMA((N_Z - 1,)),
            pltpu.SemaphoreType.DMA((N_Z - 1,)),
            pltpu.SemaphoreType.DMA((2 * N_Z,)),
            pltpu.SemaphoreType.DMA((2 * N_Z,)),
            pltpu.SemaphoreType.DMA((2 * N_Z,)),
            pltpu.SemaphoreType.DMA((2 * N_Z,)),
            pltpu.SemaphoreType.DMA,
        ],
        compiler_params=pltpu.CompilerParams(
            collective_id=0, vmem_limit_bytes=100 * 1024 * 1024
        ),
    )(partial, resid, gamma2d)
